# Initial kernel scaffold; baseline (speedup 1.0000x reference)
import jax
import jax.numpy as jnp
from jax.experimental import pallas as pl


def kernel(h, edge_index):
    raise NotImplementedError("write your pallas kernel here")


# speedup vs baseline: 1.2019x; 1.2019x over previous
import functools

import jax
import jax.numpy as jnp
from jax import lax
from jax.experimental import pallas as pl
from jax.experimental.pallas import tpu as pltpu
from jax.experimental.pallas import tpu_sc as plsc

E = 320000
D = 128
N = 10000

_info = plsc.get_sparse_core_info()
NC, NS, L = _info.num_cores, _info.num_subcores, _info.num_lanes
NW = NC * NS
E_PER_W = E // NW
C = 400
N_CHUNKS = E_PER_W // C
G = C // 16


def _make_kernel():
    mesh = plsc.VectorSubcoreMesh(core_axis_name="c", subcore_axis_name="s")

    @functools.partial(
        pl.kernel,
        mesh=mesh,
        out_type=jax.ShapeDtypeStruct((E,), jnp.float32),
        compiler_params=pltpu.CompilerParams(needs_layout_passes=False),
        scratch_types=[
            pltpu.VMEM((C,), jnp.int32),
            pltpu.VMEM((C,), jnp.int32),
            pltpu.VMEM((C, D), jnp.float32),
            pltpu.VMEM((C, D), jnp.float32),
            pltpu.VMEM((C,), jnp.float32),
            pltpu.SemaphoreType.DMA,
            pltpu.SemaphoreType.DMA,
        ],
    )
    def edge_dot(h_hbm, src_hbm, dst_hbm, out_hbm,
                 src_idx, dst_idx, src_rows, dst_rows, out_v, sem0, sem1):
        wid = lax.axis_index("s") * NC + lax.axis_index("c")
        base = wid * E_PER_W
        lanes = lax.iota(jnp.int32, L)

        def chunk_body(c, carry):
            off = base + c * C
            pltpu.sync_copy(src_hbm.at[pl.ds(off, C)], src_idx)
            pltpu.sync_copy(dst_hbm.at[pl.ds(off, C)], dst_idx)
            cp0 = pltpu.async_copy(h_hbm.at[src_idx], src_rows, sem0)
            cp1 = pltpu.async_copy(h_hbm.at[dst_idx], dst_rows, sem1)
            cp0.wait()
            cp1.wait()

            def group_body(g, carry2):
                rows = g * L + lanes
                acc = jnp.zeros((L,), jnp.float32)
                for d in range(D):
                    col = jnp.full((L,), d, jnp.int32)
                    s = plsc.load_gather(src_rows, [rows, col])
                    t = plsc.load_gather(dst_rows, [rows, col])
                    acc = acc + s * t
                out_v[pl.ds(g * L, L)] = acc
                return carry2

            lax.fori_loop(0, G, group_body, 0)
            pltpu.sync_copy(out_v, out_hbm.at[pl.ds(off, C)])
            return carry

        lax.fori_loop(0, N_CHUNKS, chunk_body, 0)

    return edge_dot


_edge_dot = _make_kernel()


def kernel(h, edge_index):
    src = edge_index[0].astype(jnp.int32)
    dst = edge_index[1].astype(jnp.int32)
    scores = _edge_dot(h, src, dst)
    return scores.reshape(E, 1)


# speedup vs baseline: 3.6845x; 3.0656x over previous
import functools

import jax
import jax.numpy as jnp
from jax import lax
from jax.experimental import pallas as pl
from jax.experimental.pallas import tpu as pltpu
from jax.experimental.pallas import tpu_sc as plsc

E = 320000
D = 128
N = 10000

_info = plsc.get_sparse_core_info()
NC, NS, L = _info.num_cores, _info.num_subcores, _info.num_lanes
F = D // NS
E_CORE = E // NC
C_E = 6400
N_CH = E_CORE // C_E
G = C_E // L


def _make_sc_kernel():
    mesh = plsc.VectorSubcoreMesh(core_axis_name="c", subcore_axis_name="s")

    @functools.partial(
        pl.kernel,
        mesh=mesh,
        out_type=jax.ShapeDtypeStruct((NS, E), jnp.float32),
        compiler_params=pltpu.CompilerParams(
            needs_layout_passes=False, use_tc_tiling_on_sc=False
        ),
        scratch_types=[
            pltpu.VMEM((N, F), jnp.float32),
            pltpu.VMEM((C_E,), jnp.int32),
            pltpu.VMEM((C_E,), jnp.int32),
            pltpu.VMEM((C_E,), jnp.float32),
        ],
    )
    def edge_partial(h_hbm, src_hbm, dst_hbm, part_hbm,
                     table_v, src_idx, dst_idx, partial_v):
        core = lax.axis_index("c")
        s = lax.axis_index("s")
        pltpu.sync_copy(h_hbm.at[:, pl.ds(s * F, F)], table_v)
        base = core * E_CORE

        def chunk_body(ch, carry):
            off = base + ch * C_E
            pltpu.sync_copy(src_hbm.at[pl.ds(off, C_E)], src_idx)
            pltpu.sync_copy(dst_hbm.at[pl.ds(off, C_E)], dst_idx)

            @plsc.parallel_loop(0, G, unroll=4)
            def group_body(g):
                sv = src_idx[pl.ds(g * L, L)]
                dv = dst_idx[pl.ds(g * L, L)]
                prods = []
                for f in range(F):
                    fc = jnp.full((L,), f, jnp.int32)
                    a = plsc.load_gather(table_v, [sv, fc])
                    b = plsc.load_gather(table_v, [dv, fc])
                    prods.append(a * b)
                while len(prods) > 1:
                    prods = [x + y for x, y in zip(prods[::2], prods[1::2])]
                partial_v[pl.ds(g * L, L)] = prods[0]
            pltpu.sync_copy(partial_v, part_hbm.at[s, pl.ds(off, C_E)])
            return carry

        lax.fori_loop(0, N_CH, chunk_body, 0)

    return edge_partial


_edge_partial = _make_sc_kernel()

def _reduce_body(p_ref, o_ref):
    o_ref[...] = jnp.sum(p_ref[...], axis=0)


_reduce = pl.pallas_call(
    _reduce_body,
    out_shape=jax.ShapeDtypeStruct((E,), jnp.float32),
)


def kernel(h, edge_index):
    src = edge_index[0].astype(jnp.int32)
    dst = edge_index[1].astype(jnp.int32)
    partials = _edge_partial(h, src, dst)
    return _reduce(partials).reshape(E, 1)


# speedup vs baseline: 5.1609x; 1.4007x over previous
import functools

import jax
import jax.numpy as jnp
from jax import lax
from jax.experimental import pallas as pl
from jax.experimental.pallas import tpu as pltpu
from jax.experimental.pallas import tpu_sc as plsc

E = 320000
D = 128
N = 10000

_info = plsc.get_sparse_core_info()
NC, NS, L = _info.num_cores, _info.num_subcores, _info.num_lanes
F = D // NS
E_CORE = E // NC
C_E = 6400
N_CH = E_CORE // C_E
G = C_E // L


def _make_sc_kernel():
    mesh = plsc.VectorSubcoreMesh(core_axis_name="c", subcore_axis_name="s")

    @functools.partial(
        pl.kernel,
        mesh=mesh,
        out_type=jax.ShapeDtypeStruct((NS, E), jnp.float32),
        compiler_params=pltpu.CompilerParams(
            needs_layout_passes=False, use_tc_tiling_on_sc=False
        ),
        scratch_types=[
            pltpu.VMEM((F, N), jnp.float32),
            pltpu.VMEM((C_E,), jnp.int32),
            pltpu.VMEM((C_E,), jnp.int32),
            pltpu.VMEM((C_E,), jnp.float32),
        ],
    )
    def edge_partial(ht_hbm, src_hbm, dst_hbm, part_hbm,
                     table_v, src_idx, dst_idx, partial_v):
        core = lax.axis_index("c")
        s = lax.axis_index("s")
        pltpu.sync_copy(ht_hbm.at[pl.ds(s * F, F), :], table_v)
        base = core * E_CORE

        def chunk_body(ch, carry):
            off = base + ch * C_E
            pltpu.sync_copy(src_hbm.at[pl.ds(off, C_E)], src_idx)
            pltpu.sync_copy(dst_hbm.at[pl.ds(off, C_E)], dst_idx)

            @plsc.parallel_loop(0, G, unroll=4)
            def group_body(g):
                sv = src_idx[pl.ds(g * L, L)]
                dv = dst_idx[pl.ds(g * L, L)]
                prods = []
                for f in range(F):
                    fc = jnp.full((L,), f, jnp.int32)
                    a = plsc.load_gather(table_v, [fc, sv])
                    b = plsc.load_gather(table_v, [fc, dv])
                    prods.append(a * b)
                while len(prods) > 1:
                    prods = [x + y for x, y in zip(prods[::2], prods[1::2])]
                partial_v[pl.ds(g * L, L)] = prods[0]
            pltpu.sync_copy(partial_v, part_hbm.at[s, pl.ds(off, C_E)])
            return carry

        lax.fori_loop(0, N_CH, chunk_body, 0)

    return edge_partial


_edge_partial = _make_sc_kernel()

def _reduce_body(p_ref, o_ref):
    o_ref[...] = jnp.sum(p_ref[...], axis=0)


_reduce = pl.pallas_call(
    _reduce_body,
    out_shape=jax.ShapeDtypeStruct((E,), jnp.float32),
)


def kernel(h, edge_index):
    src = edge_index[0].astype(jnp.int32)
    dst = edge_index[1].astype(jnp.int32)
    partials = _edge_partial(h.T, src, dst)
    return _reduce(partials).reshape(E, 1)


# speedup vs baseline: 6.2180x; 1.2048x over previous
import functools

import jax
import jax.numpy as jnp
from jax import lax
from jax.experimental import pallas as pl
from jax.experimental.pallas import tpu as pltpu
from jax.experimental.pallas import tpu_sc as plsc

E = 320000
D = 128
N = 10000

_info = plsc.get_sparse_core_info()
NC, NS, L = _info.num_cores, _info.num_subcores, _info.num_lanes
F = D // NS
E_CORE = E // NC
C_E = 6400
N_CH = E_CORE // C_E
G = C_E // L


def _make_sc_kernel():
    mesh = plsc.VectorSubcoreMesh(core_axis_name="c", subcore_axis_name="s")

    @functools.partial(
        pl.kernel,
        mesh=mesh,
        out_type=jax.ShapeDtypeStruct((NS, E), jnp.float32),
        compiler_params=pltpu.CompilerParams(
            needs_layout_passes=False, use_tc_tiling_on_sc=False
        ),
        scratch_types=[
            pltpu.VMEM((F, N), jnp.float32),
            pltpu.VMEM((2, C_E), jnp.int32),
            pltpu.VMEM((2, C_E), jnp.int32),
            pltpu.VMEM((2, C_E), jnp.float32),
            pltpu.SemaphoreType.DMA((2,)),
            pltpu.SemaphoreType.DMA((2,)),
        ],
    )
    def edge_partial(ht_hbm, src_hbm, dst_hbm, part_hbm,
                     table_v, src_idx, dst_idx, partial_v, sem_idx, sem_pw):
        core = lax.axis_index("c")
        s = lax.axis_index("s")
        base = core * E_CORE

        def start_idx(c, b):
            off = base + c * C_E
            pltpu.async_copy(src_hbm.at[pl.ds(off, C_E)], src_idx.at[b],
                             sem_idx.at[b])
            pltpu.async_copy(dst_hbm.at[pl.ds(off, C_E)], dst_idx.at[b],
                             sem_idx.at[b])

        def wait_idx(c, b):
            off = base + c * C_E
            pltpu.make_async_copy(src_hbm.at[pl.ds(off, C_E)], src_idx.at[b],
                                  sem_idx.at[b]).wait()
            pltpu.make_async_copy(dst_hbm.at[pl.ds(off, C_E)], dst_idx.at[b],
                                  sem_idx.at[b]).wait()

        def wait_pw(c, b):
            off = base + c * C_E
            pltpu.make_async_copy(partial_v.at[b],
                                  part_hbm.at[s, pl.ds(off, C_E)],
                                  sem_pw.at[b]).wait()

        start_idx(0, 0)
        pltpu.sync_copy(ht_hbm.at[pl.ds(s * F, F), :], table_v)

        def chunk_body(c, carry):
            b = lax.rem(c, 2)
            wait_idx(c, b)

            @pl.when(c + 1 < N_CH)
            def _():
                start_idx(c + 1, 1 - b)

            @pl.when(c >= 2)
            def _():
                wait_pw(c - 2, b)

            @plsc.parallel_loop(0, G, unroll=4)
            def group_body(g):
                sv = src_idx[b, pl.ds(g * L, L)]
                dv = dst_idx[b, pl.ds(g * L, L)]
                prods = []
                for f in range(F):
                    fc = jnp.full((L,), f, jnp.int32)
                    a = plsc.load_gather(table_v, [fc, sv])
                    bb = plsc.load_gather(table_v, [fc, dv])
                    prods.append(a * bb)
                while len(prods) > 1:
                    prods = [x + y for x, y in zip(prods[::2], prods[1::2])]
                partial_v[b, pl.ds(g * L, L)] = prods[0]

            off = base + c * C_E
            pltpu.async_copy(partial_v.at[b], part_hbm.at[s, pl.ds(off, C_E)],
                             sem_pw.at[b])
            return carry

        lax.fori_loop(0, N_CH, chunk_body, 0)
        for c in (N_CH - 2, N_CH - 1):
            wait_pw(c, c % 2)

    return edge_partial


_edge_partial = _make_sc_kernel()


def _reduce_body(p_ref, o_ref):
    o_ref[...] = jnp.sum(p_ref[...], axis=0)


_reduce = pl.pallas_call(
    _reduce_body,
    out_shape=jax.ShapeDtypeStruct((E,), jnp.float32),
)


def kernel(h, edge_index):
    src = edge_index[0].astype(jnp.int32)
    dst = edge_index[1].astype(jnp.int32)
    partials = _edge_partial(h.T, src, dst)
    return _reduce(partials).reshape(E, 1)


# speedup vs baseline: 7.5179x; 1.2091x over previous
import functools

import jax
import jax.numpy as jnp
from jax import lax
from jax.experimental import pallas as pl
from jax.experimental.pallas import tpu as pltpu
from jax.experimental.pallas import tpu_sc as plsc

E = 320000
D = 128
N = 10000

_info = plsc.get_sparse_core_info()
NC, NS, L = _info.num_cores, _info.num_subcores, _info.num_lanes
NT = 8
NEG = 4
P = D // 2 // NT
E_GRP = E // NEG
C_E = 4000
N_CH = E_GRP // C_E
G = C_E // L
_MASKHI = -65536


def _pack_body(ht_ref, o_ref):
    x = ht_ref[...]
    u16 = lax.bitcast_convert_type(x.astype(jnp.bfloat16), jnp.uint16)
    u = u16.astype(jnp.int32).reshape(NT * P, 2, N)
    o_ref[...] = (u[:, 0, :] << 16) | u[:, 1, :]


_pack = pl.pallas_call(
    _pack_body,
    out_shape=jax.ShapeDtypeStruct((NT * P, N), jnp.int32),
)


def _make_sc_kernel():
    mesh = plsc.VectorSubcoreMesh(core_axis_name="c", subcore_axis_name="s")

    @functools.partial(
        pl.kernel,
        mesh=mesh,
        out_type=jax.ShapeDtypeStruct((NT, E), jnp.float32),
        compiler_params=pltpu.CompilerParams(
            needs_layout_passes=False, use_tc_tiling_on_sc=False
        ),
        scratch_types=[
            pltpu.VMEM((P, N), jnp.int32),
            pltpu.VMEM((2, C_E), jnp.int32),
            pltpu.VMEM((2, C_E), jnp.int32),
            pltpu.VMEM((2, C_E), jnp.float32),
            pltpu.SemaphoreType.DMA((2,)),
            pltpu.SemaphoreType.DMA((2,)),
        ],
    )
    def edge_partial(hp_hbm, src_hbm, dst_hbm, part_hbm,
                     table_v, src_idx, dst_idx, partial_v, sem_idx, sem_pw):
        core = lax.axis_index("c")
        s = lax.axis_index("s")
        t = lax.rem(s, NT)
        eg = core * 2 + s // NT
        base = eg * E_GRP

        def start_idx(c, b):
            off = base + c * C_E
            pltpu.async_copy(src_hbm.at[pl.ds(off, C_E)], src_idx.at[b],
                             sem_idx.at[b])
            pltpu.async_copy(dst_hbm.at[pl.ds(off, C_E)], dst_idx.at[b],
                             sem_idx.at[b])

        def wait_idx(c, b):
            off = base + c * C_E
            pltpu.make_async_copy(src_hbm.at[pl.ds(off, C_E)], src_idx.at[b],
                                  sem_idx.at[b]).wait()
            pltpu.make_async_copy(dst_hbm.at[pl.ds(off, C_E)], dst_idx.at[b],
                                  sem_idx.at[b]).wait()

        def wait_pw(c, b):
            off = base + c * C_E
            pltpu.make_async_copy(partial_v.at[b],
                                  part_hbm.at[t, pl.ds(off, C_E)],
                                  sem_pw.at[b]).wait()

        start_idx(0, 0)
        pltpu.sync_copy(hp_hbm.at[pl.ds(t * P, P), :], table_v)

        def chunk_body(c, carry):
            b = lax.rem(c, 2)
            wait_idx(c, b)

            @pl.when(c + 1 < N_CH)
            def _():
                start_idx(c + 1, 1 - b)

            @pl.when(c >= 2)
            def _():
                wait_pw(c - 2, b)

            @plsc.parallel_loop(0, G, unroll=4)
            def group_body(g):
                sv = src_idx[b, pl.ds(g * L, L)]
                dv = dst_idx[b, pl.ds(g * L, L)]
                prods = []
                for p in range(P):
                    pc = jnp.full((L,), p, jnp.int32)
                    ws = plsc.load_gather(table_v, [pc, sv])
                    wd = plsc.load_gather(table_v, [pc, dv])
                    hs = plsc.bitcast(ws & _MASKHI, jnp.float32)
                    hd = plsc.bitcast(wd & _MASKHI, jnp.float32)
                    ls = plsc.bitcast(ws << 16, jnp.float32)
                    ld = plsc.bitcast(wd << 16, jnp.float32)
                    prods.append(hs * hd)
                    prods.append(ls * ld)
                while len(prods) > 1:
                    prods = [x + y for x, y in zip(prods[::2], prods[1::2])]
                partial_v[b, pl.ds(g * L, L)] = prods[0]

            off = base + c * C_E
            pltpu.async_copy(partial_v.at[b], part_hbm.at[t, pl.ds(off, C_E)],
                             sem_pw.at[b])
            return carry

        lax.fori_loop(0, N_CH, chunk_body, 0)
        for c in (N_CH - 2, N_CH - 1):
            wait_pw(c, c % 2)

    return edge_partial


_edge_partial = _make_sc_kernel()


def _reduce_body(p_ref, o_ref):
    o_ref[...] = jnp.sum(p_ref[...], axis=0)


_reduce = pl.pallas_call(
    _reduce_body,
    out_shape=jax.ShapeDtypeStruct((E,), jnp.float32),
)


def kernel(h, edge_index):
    src = edge_index[0].astype(jnp.int32)
    dst = edge_index[1].astype(jnp.int32)
    partials = _edge_partial(_pack(h.T), src, dst)
    return _reduce(partials).reshape(E, 1)


# speedup vs baseline: 9.7530x; 1.2973x over previous
import functools

import jax
import jax.numpy as jnp
from jax import lax
from jax.experimental import pallas as pl
from jax.experimental.pallas import tpu as pltpu
from jax.experimental.pallas import tpu_sc as plsc

E = 320000
D = 128
N = 10000

_info = plsc.get_sparse_core_info()
NC, NS, L = _info.num_cores, _info.num_subcores, _info.num_lanes
NT = 8
NEG = 4
P = D // 2 // NT
E_GRP = E // NEG
C_E = 4000
N_CH = E_GRP // C_E
G = C_E // L
_MASKHI = -65536


def _pack_body(ht_ref, o_ref):
    x = ht_ref[...]
    u16 = lax.bitcast_convert_type(x.astype(jnp.bfloat16), jnp.uint16)
    u = u16.astype(jnp.int32).reshape(NT * P, 2, N)
    o_ref[...] = (u[:, 0, :] << 16) | u[:, 1, :]


_pack = pl.pallas_call(
    _pack_body,
    out_shape=jax.ShapeDtypeStruct((NT * P, N), jnp.int32),
)


def _make_sc_kernel():
    mesh = plsc.VectorSubcoreMesh(core_axis_name="c", subcore_axis_name="s")

    @functools.partial(
        pl.kernel,
        mesh=mesh,
        out_type=jax.ShapeDtypeStruct((NT, E), jnp.float32),
        compiler_params=pltpu.CompilerParams(
            needs_layout_passes=False, use_tc_tiling_on_sc=False
        ),
        scratch_types=[
            pltpu.VMEM((P, N), jnp.int32),
            pltpu.VMEM((2, C_E), jnp.int32),
            pltpu.VMEM((2, C_E), jnp.int32),
            pltpu.VMEM((2, C_E), jnp.float32),
            pltpu.SemaphoreType.DMA((2,)),
            pltpu.SemaphoreType.DMA((2,)),
        ],
    )
    def edge_partial(hp_hbm, src_hbm, dst_hbm, part_hbm,
                     table_v, src_idx, dst_idx, partial_v, sem_idx, sem_pw):
        core = lax.axis_index("c")
        s = lax.axis_index("s")
        t = lax.rem(s, NT)
        eg = core * 2 + s // NT
        base = eg * E_GRP

        def start_idx(c, b):
            off = base + c * C_E
            pltpu.async_copy(src_hbm.at[pl.ds(off, C_E)], src_idx.at[b],
                             sem_idx.at[b])
            pltpu.async_copy(dst_hbm.at[pl.ds(off, C_E)], dst_idx.at[b],
                             sem_idx.at[b])

        def wait_idx(c, b):
            off = base + c * C_E
            pltpu.make_async_copy(src_hbm.at[pl.ds(off, C_E)], src_idx.at[b],
                                  sem_idx.at[b]).wait()
            pltpu.make_async_copy(dst_hbm.at[pl.ds(off, C_E)], dst_idx.at[b],
                                  sem_idx.at[b]).wait()

        def wait_pw(c, b):
            off = base + c * C_E
            pltpu.make_async_copy(partial_v.at[b],
                                  part_hbm.at[t, pl.ds(off, C_E)],
                                  sem_pw.at[b]).wait()

        start_idx(0, 0)
        pltpu.sync_copy(hp_hbm.at[pl.ds(t * P, P), :], table_v)

        def chunk_body(c, carry):
            b = lax.rem(c, 2)
            wait_idx(c, b)

            @pl.when(c + 1 < N_CH)
            def _():
                start_idx(c + 1, 1 - b)

            @pl.when(c >= 2)
            def _():
                wait_pw(c - 2, b)

            @plsc.parallel_loop(0, G, unroll=4)
            def group_body(g):
                sv = src_idx[b, pl.ds(g * L, L)]
                dv = dst_idx[b, pl.ds(g * L, L)]
                prods = []
                for p in range(P):
                    pc = jnp.full((L,), p, jnp.int32)
                    ws = plsc.load_gather(table_v, [pc, sv])
                    wd = plsc.load_gather(table_v, [pc, dv])
                    sb = plsc.bitcast(ws, jnp.bfloat16)
                    db = plsc.bitcast(wd, jnp.bfloat16)
                    prods.append(sb * db)
                while len(prods) > 1:
                    prods = [x + y for x, y in zip(prods[::2], prods[1::2])]
                accw = plsc.bitcast(prods[0], jnp.int32)
                hi = plsc.bitcast(accw & _MASKHI, jnp.float32)
                lo = plsc.bitcast(accw << 16, jnp.float32)
                partial_v[b, pl.ds(g * L, L)] = hi + lo

            off = base + c * C_E
            pltpu.async_copy(partial_v.at[b], part_hbm.at[t, pl.ds(off, C_E)],
                             sem_pw.at[b])
            return carry

        lax.fori_loop(0, N_CH, chunk_body, 0)
        for c in (N_CH - 2, N_CH - 1):
            wait_pw(c, c % 2)

    return edge_partial


_edge_partial = _make_sc_kernel()


def _reduce_body(p_ref, o_ref):
    o_ref[...] = jnp.sum(p_ref[...], axis=0)


_reduce = pl.pallas_call(
    _reduce_body,
    out_shape=jax.ShapeDtypeStruct((E,), jnp.float32),
)


def kernel(h, edge_index):
    src = edge_index[0].astype(jnp.int32)
    dst = edge_index[1].astype(jnp.int32)
    partials = _edge_partial(_pack(h.T), src, dst)
    return _reduce(partials).reshape(E, 1)


# speedup vs baseline: 10.1575x; 1.0415x over previous
import functools

import jax
import jax.numpy as jnp
from jax import lax
from jax.experimental import pallas as pl
from jax.experimental.pallas import tpu as pltpu
from jax.experimental.pallas import tpu_sc as plsc

E = 320000
D = 128
N = 10000

_info = plsc.get_sparse_core_info()
NC, NS, L = _info.num_cores, _info.num_subcores, _info.num_lanes
NT = 8
NEG = 4
P = D // 2 // NT
E_GRP = E // NEG
C_E = 4000
N_CH = E_GRP // C_E
G = C_E // L
R = C_E // L
_MASKHI = -65536


def _pack_body(ht_ref, o_ref):
    x = ht_ref[...]
    u16 = lax.bitcast_convert_type(x.astype(jnp.bfloat16), jnp.uint16)
    u = u16.astype(jnp.int32).reshape(NT * P, 2, N)
    o_ref[...] = (u[:, 0, :] << 16) | u[:, 1, :]


_pack = pl.pallas_call(
    _pack_body,
    out_shape=jax.ShapeDtypeStruct((NT * P, N), jnp.int32),
)


def _make_sc_kernel():
    mesh = plsc.VectorSubcoreMesh(core_axis_name="c", subcore_axis_name="s")

    @functools.partial(
        pl.kernel,
        mesh=mesh,
        out_type=jax.ShapeDtypeStruct((E // L, L), jnp.float32),
        compiler_params=pltpu.CompilerParams(
            needs_layout_passes=False, use_tc_tiling_on_sc=False
        ),
        scratch_types=[
            pltpu.VMEM((P, N), jnp.int32),
            pltpu.VMEM((2, C_E), jnp.int32),
            pltpu.VMEM((2, C_E), jnp.int32),
            pltpu.VMEM((2, R, L), jnp.float32),
            pltpu.VMEM((4, R), jnp.int32),
            pltpu.VMEM_SHARED((4 * R, L), jnp.float32),
            pltpu.SemaphoreType.DMA((2,)),
            pltpu.SemaphoreType.DMA((2,)),
        ],
    )
    def edge_score(hp_hbm, src_hbm, dst_hbm, rows_hbm, out_hbm,
                   table_v, src_idx, dst_idx, partial_v, rows_v, acc_sh,
                   sem_idx, sem_out):
        core = lax.axis_index("c")
        s = lax.axis_index("s")
        t = lax.rem(s, NT)
        leg = s // NT
        eg = core * 2 + leg
        slot = leg * 2
        base = eg * E_GRP

        def start_idx(c, b):
            off = base + c * C_E
            pltpu.async_copy(src_hbm.at[pl.ds(off, C_E)], src_idx.at[b],
                             sem_idx.at[b])
            pltpu.async_copy(dst_hbm.at[pl.ds(off, C_E)], dst_idx.at[b],
                             sem_idx.at[b])

        def wait_idx(c, b):
            off = base + c * C_E
            pltpu.make_async_copy(src_hbm.at[pl.ds(off, C_E)], src_idx.at[b],
                                  sem_idx.at[b]).wait()
            pltpu.make_async_copy(dst_hbm.at[pl.ds(off, C_E)], dst_idx.at[b],
                                  sem_idx.at[b]).wait()

        def out_rows(c):
            return eg * (E_GRP // L) + c * R

        def wait_out(c, b):
            pltpu.make_async_copy(
                acc_sh.at[pl.ds((slot + b) * R, R), :],
                out_hbm.at[pl.ds(out_rows(c), R), :],
                sem_out.at[b]).wait()

        start_idx(0, 0)
        pltpu.sync_copy(rows_hbm, rows_v)
        pltpu.sync_copy(hp_hbm.at[pl.ds(t * P, P), :], table_v)

        def chunk_body(c, carry):
            b = lax.rem(c, 2)
            wait_idx(c, b)

            @pl.when(c + 1 < N_CH)
            def _():
                start_idx(c + 1, 1 - b)

            @plsc.parallel_loop(0, G, unroll=4)
            def group_body(g):
                sv = src_idx[b, pl.ds(g * L, L)]
                dv = dst_idx[b, pl.ds(g * L, L)]
                prods = []
                for p in range(P):
                    pc = jnp.full((L,), p, jnp.int32)
                    ws = plsc.load_gather(table_v, [pc, sv])
                    wd = plsc.load_gather(table_v, [pc, dv])
                    sb = plsc.bitcast(ws, jnp.bfloat16)
                    db = plsc.bitcast(wd, jnp.bfloat16)
                    prods.append(sb * db)
                while len(prods) > 1:
                    prods = [x + y for x, y in zip(prods[::2], prods[1::2])]
                accw = plsc.bitcast(prods[0], jnp.int32)
                hi = plsc.bitcast(accw & _MASKHI, jnp.float32)
                lo = plsc.bitcast(accw << 16, jnp.float32)
                partial_v[b, g] = hi + lo

            @pl.when(jnp.logical_and(t == 0, c >= 2))
            def _():
                wait_out(c - 2, b)

            @pl.when(t == 0)
            def _():
                pltpu.sync_copy(partial_v.at[b],
                                acc_sh.at[pl.ds((slot + b) * R, R), :])

            plsc.subcore_barrier()

            @pl.when(t > 0)
            def _():
                pltpu.sync_copy(partial_v.at[b],
                                acc_sh.at[rows_v.at[slot + b]], add=True)

            plsc.subcore_barrier()

            @pl.when(t == 0)
            def _():
                pltpu.async_copy(acc_sh.at[pl.ds((slot + b) * R, R), :],
                                 out_hbm.at[pl.ds(out_rows(c), R), :],
                                 sem_out.at[b])
            return carry

        lax.fori_loop(0, N_CH, chunk_body, 0)

        @pl.when(t == 0)
        def _():
            for c in (N_CH - 2, N_CH - 1):
                wait_out(c, c % 2)

    return edge_score


_edge_score = _make_sc_kernel()

import numpy as _np

_ROWS = _np.arange(4 * R, dtype=_np.int32).reshape(4, R)


def kernel(h, edge_index):
    src = edge_index[0].astype(jnp.int32)
    dst = edge_index[1].astype(jnp.int32)
    scores = _edge_score(_pack(h.T), src, dst, _ROWS)
    return scores.reshape(E, 1)
